# baseline (speedup 1.0000x reference)
import functools

import jax
import jax.numpy as jnp
from jax import lax
from jax.experimental import pallas as pl
from jax.experimental.pallas import tpu as pltpu
from jax.experimental.pallas import tpu_sc as plsc

ROWS = 16384
COLS = 1024
L = 16

ROWS_SC = 5120
ROWS_TC = ROWS - ROWS_SC

NC = 2
NS = 16
NW = NC * NS
RPW = ROWS_SC // NW
G = L
NGS = RPW // G
NQ = 4
QW = COLS // NQ

_mesh = plsc.VectorSubcoreMesh(core_axis_name="c", subcore_axis_name="s")


@functools.partial(
    pl.kernel,
    out_type=jax.ShapeDtypeStruct((ROWS_SC,), jnp.int32),
    mesh=_mesh,
    scratch_types=[
        pltpu.VMEM((2, G, COLS), jnp.float32),
        pltpu.VMEM((RPW,), jnp.int32),
        pltpu.SemaphoreType.DMA,
        pltpu.SemaphoreType.DMA,
    ],
    compiler_params=pltpu.CompilerParams(
        use_tc_tiling_on_sc=True, needs_layout_passes=False
    ),
)
def _argmax_sc(emission_hbm, out_hbm, buf, outv, sem0, sem1):
    wid = lax.axis_index("s") * NC + lax.axis_index("c")
    row0 = wid * RPW
    sems = (sem0, sem1)
    liota = lax.iota(jnp.int32, L)

    def chunk_dma(g, b):
        return pltpu.make_async_copy(
            emission_hbm.at[pl.ds(row0 + g * G, G), :], buf.at[b], sems[b]
        )

    def compute(g, b):
        acc = jnp.zeros((L,), jnp.int32)
        for r in range(G):
            init = ()
            for _ in range(NQ):
                init = init + (
                    jnp.full((L,), -jnp.inf, jnp.float32),
                    jnp.zeros((L,), jnp.int32),
                )

            def kbody(k, carry):
                ksplat = jnp.full((L,), k, jnp.int32)
                out = ()
                for q in range(NQ):
                    mv, mk = carry[2 * q], carry[2 * q + 1]
                    v = buf[b, r, pl.ds(k * L + q * QW, L)]
                    take = v > mv
                    out = out + (
                        jnp.where(take, v, mv),
                        jnp.where(take, ksplat, mk),
                    )
                return out

            carry = plsc.parallel_loop(0, QW // L, unroll=4, carry=init)(kbody)
            mv = carry[0]
            mi = (carry[1] << 4) + liota
            for q in range(1, NQ):
                v = carry[2 * q]
                i = (carry[2 * q + 1] << 4) + liota + q * QW
                take = v > mv
                mv = jnp.where(take, v, mv)
                mi = jnp.where(take, i, mi)
            m = jax.lax.reduce_max(mv, axes=(0,))
            masked = jnp.where(mv == m, mi, COLS)
            best = jax.lax.reduce_min(masked, axes=(0,))
            acc = jnp.where(liota == r, jnp.full((L,), best, jnp.int32), acc)
        outv[pl.ds(g * G, G)] = acc

    chunk_dma(0, 0).start()

    def outer(i, _):
        g0 = 2 * i
        chunk_dma(g0 + 1, 1).start()
        chunk_dma(g0, 0).wait()
        compute(g0, 0)

        @pl.when(g0 + 2 < NGS)
        def _():
            chunk_dma(g0 + 2, 0).start()

        chunk_dma(g0 + 1, 1).wait()
        compute(g0 + 1, 1)
        return 0

    lax.fori_loop(0, NGS // 2, outer, 0)
    pltpu.sync_copy(outv, out_hbm.at[pl.ds(row0, RPW)])


BLK = 256
NBLK = ROWS_TC // BLK
NBUF = 4


def _tc_body(em_hbm, o_hbm, buf, ov, osem, *sems):
    def blk_dma(g, b):
        return pltpu.make_async_copy(
            em_hbm.at[pl.ds(ROWS_SC + g * BLK, BLK), :], buf.at[b], sems[b]
        )

    for b in range(NBUF - 1):
        blk_dma(b, b).start()

    def compute(g, b):
        nsl = COLS // 128
        xs = [buf[b, :, pl.ds(j * 128, 128)] for j in range(nsl)]
        m = xs[0]
        for j in range(1, nsl):
            m = jnp.maximum(m, xs[j])
        m = jnp.max(m, axis=-1, keepdims=True)
        li = lax.broadcasted_iota(jnp.int32, (BLK, 128), 1)
        cand = jnp.where(xs[0] == m, li, COLS)
        for j in range(1, nsl):
            cand = jnp.minimum(cand, jnp.where(xs[j] == m, li + j * 128, COLS))
        ov[pl.ds(g * BLK, BLK)] = jnp.min(cand, axis=-1)

    def outer(i, _):
        g0 = NBUF * i
        for b in range(NBUF):
            g = g0 + b

            @pl.when(g + NBUF - 1 < NBLK)
            def _():
                blk_dma(g + NBUF - 1, (b + NBUF - 1) % NBUF).start()

            blk_dma(g, b).wait()
            compute(g, b)
        return 0

    lax.fori_loop(0, NBLK // NBUF, outer, 0)
    pltpu.make_async_copy(ov, o_hbm, osem).start()
    pltpu.make_async_copy(ov, o_hbm, osem).wait()


_argmax_tc = pl.pallas_call(
    _tc_body,
    in_specs=[pl.BlockSpec(memory_space=pl.ANY)],
    out_specs=pl.BlockSpec(memory_space=pl.ANY),
    out_shape=jax.ShapeDtypeStruct((ROWS_TC,), jnp.int32),
    scratch_shapes=[
        pltpu.VMEM((NBUF, BLK, COLS), jnp.float32),
        pltpu.VMEM((ROWS_TC,), jnp.int32),
        pltpu.SemaphoreType.DMA,
    ]
    + [pltpu.SemaphoreType.DMA] * NBUF,
)


def kernel(emission, to_string):
    del to_string
    sc_out = _argmax_sc(emission)
    tc_out = _argmax_tc(emission)
    return jnp.concatenate([sc_out, tc_out])


# speedup vs baseline: 1.0438x; 1.0438x over previous
import functools

import jax
import jax.numpy as jnp
from jax import lax
from jax.experimental import pallas as pl
from jax.experimental.pallas import tpu as pltpu
from jax.experimental.pallas import tpu_sc as plsc

ROWS = 16384
COLS = 1024
L = 16

ROWS_SC = 3072
ROWS_TC = ROWS - ROWS_SC

NC = 2
NS = 16
NW = NC * NS
RPW = ROWS_SC // NW
G = L
NGS = RPW // G
NQ = 4
QW = COLS // NQ

_mesh = plsc.VectorSubcoreMesh(core_axis_name="c", subcore_axis_name="s")


@functools.partial(
    pl.kernel,
    out_type=jax.ShapeDtypeStruct((ROWS_SC,), jnp.int32),
    mesh=_mesh,
    scratch_types=[
        pltpu.VMEM((2, G, COLS), jnp.float32),
        pltpu.VMEM((RPW,), jnp.int32),
        pltpu.SemaphoreType.DMA,
        pltpu.SemaphoreType.DMA,
    ],
    compiler_params=pltpu.CompilerParams(
        use_tc_tiling_on_sc=True, needs_layout_passes=False
    ),
)
def _argmax_sc(emission_hbm, out_hbm, buf, outv, sem0, sem1):
    wid = lax.axis_index("s") * NC + lax.axis_index("c")
    row0 = wid * RPW
    sems = (sem0, sem1)
    liota = lax.iota(jnp.int32, L)

    def chunk_dma(g, b):
        return pltpu.make_async_copy(
            emission_hbm.at[pl.ds(row0 + g * G, G), :], buf.at[b], sems[b]
        )

    def compute(g, b):
        acc = jnp.zeros((L,), jnp.int32)
        for r in range(G):
            init = ()
            for _ in range(NQ):
                init = init + (
                    jnp.full((L,), -jnp.inf, jnp.float32),
                    jnp.zeros((L,), jnp.int32),
                )

            def kbody(k, carry):
                ksplat = jnp.full((L,), k, jnp.int32)
                out = ()
                for q in range(NQ):
                    mv, mk = carry[2 * q], carry[2 * q + 1]
                    v = buf[b, r, pl.ds(k * L + q * QW, L)]
                    take = v > mv
                    out = out + (
                        jnp.where(take, v, mv),
                        jnp.where(take, ksplat, mk),
                    )
                return out

            carry = plsc.parallel_loop(0, QW // L, unroll=4, carry=init)(kbody)
            mv = carry[0]
            mi = (carry[1] << 4) + liota
            for q in range(1, NQ):
                v = carry[2 * q]
                i = (carry[2 * q + 1] << 4) + liota + q * QW
                take = v > mv
                mv = jnp.where(take, v, mv)
                mi = jnp.where(take, i, mi)
            m = jax.lax.reduce_max(mv, axes=(0,))
            masked = jnp.where(mv == m, mi, COLS)
            best = jax.lax.reduce_min(masked, axes=(0,))
            acc = jnp.where(liota == r, jnp.full((L,), best, jnp.int32), acc)
        outv[pl.ds(g * G, G)] = acc

    chunk_dma(0, 0).start()

    def outer(i, _):
        g0 = 2 * i
        chunk_dma(g0 + 1, 1).start()
        chunk_dma(g0, 0).wait()
        compute(g0, 0)

        @pl.when(g0 + 2 < NGS)
        def _():
            chunk_dma(g0 + 2, 0).start()

        chunk_dma(g0 + 1, 1).wait()
        compute(g0 + 1, 1)
        return 0

    lax.fori_loop(0, NGS // 2, outer, 0)
    pltpu.sync_copy(outv, out_hbm.at[pl.ds(row0, RPW)])


BLK = 256
NBLK = ROWS_TC // BLK
NBUF = 4


def _tc_body(em_hbm, o_hbm, buf, ov, osem, *sems):
    def blk_dma(g, b):
        return pltpu.make_async_copy(
            em_hbm.at[pl.ds(ROWS_SC + g * BLK, BLK), :], buf.at[b], sems[b]
        )

    for b in range(NBUF - 1):
        blk_dma(b, b).start()

    def compute(g, b):
        nsl = COLS // 128
        xs = [buf[b, :, pl.ds(j * 128, 128)] for j in range(nsl)]
        m = xs[0]
        for j in range(1, nsl):
            m = jnp.maximum(m, xs[j])
        m = jnp.max(m, axis=-1, keepdims=True)
        li = lax.broadcasted_iota(jnp.int32, (BLK, 128), 1)
        cand = jnp.where(xs[0] == m, li, COLS)
        for j in range(1, nsl):
            cand = jnp.minimum(cand, jnp.where(xs[j] == m, li + j * 128, COLS))
        ov[pl.ds(g * BLK, BLK)] = jnp.min(cand, axis=-1)

    def outer(i, _):
        g0 = NBUF * i
        for b in range(NBUF):
            g = g0 + b

            @pl.when(g + NBUF - 1 < NBLK)
            def _():
                blk_dma(g + NBUF - 1, (b + NBUF - 1) % NBUF).start()

            blk_dma(g, b).wait()
            compute(g, b)
        return 0

    lax.fori_loop(0, NBLK // NBUF, outer, 0)
    pltpu.make_async_copy(ov, o_hbm, osem).start()
    pltpu.make_async_copy(ov, o_hbm, osem).wait()


_argmax_tc = pl.pallas_call(
    _tc_body,
    in_specs=[pl.BlockSpec(memory_space=pl.ANY)],
    out_specs=pl.BlockSpec(memory_space=pl.ANY),
    out_shape=jax.ShapeDtypeStruct((ROWS_TC,), jnp.int32),
    scratch_shapes=[
        pltpu.VMEM((NBUF, BLK, COLS), jnp.float32),
        pltpu.VMEM((ROWS_TC,), jnp.int32),
        pltpu.SemaphoreType.DMA,
    ]
    + [pltpu.SemaphoreType.DMA] * NBUF,
)


def kernel(emission, to_string):
    del to_string
    sc_out = _argmax_sc(emission)
    tc_out = _argmax_tc(emission)
    return jnp.concatenate([sc_out, tc_out])


# speedup vs baseline: 1.0474x; 1.0034x over previous
import functools

import jax
import jax.numpy as jnp
from jax import lax
from jax.experimental import pallas as pl
from jax.experimental.pallas import tpu as pltpu
from jax.experimental.pallas import tpu_sc as plsc

ROWS = 16384
COLS = 1024
L = 16

ROWS_SC = 3072
ROWS_TC = ROWS - ROWS_SC

NC = 2
NS = 16
NW = NC * NS
RPW = ROWS_SC // NW
G = L
NGS = RPW // G
NQ = 4
QW = COLS // NQ

_mesh = plsc.VectorSubcoreMesh(core_axis_name="c", subcore_axis_name="s")


@functools.partial(
    pl.kernel,
    out_type=jax.ShapeDtypeStruct((ROWS_SC,), jnp.int32),
    mesh=_mesh,
    scratch_types=[
        pltpu.VMEM((2, G, COLS), jnp.float32),
        pltpu.VMEM((RPW,), jnp.int32),
        pltpu.SemaphoreType.DMA,
        pltpu.SemaphoreType.DMA,
    ],
    compiler_params=pltpu.CompilerParams(
        use_tc_tiling_on_sc=True, needs_layout_passes=False
    ),
)
def _argmax_sc(emission_hbm, out_hbm, buf, outv, sem0, sem1):
    wid = lax.axis_index("s") * NC + lax.axis_index("c")
    row0 = wid * RPW
    sems = (sem0, sem1)
    liota = lax.iota(jnp.int32, L)

    def chunk_dma(g, b):
        return pltpu.make_async_copy(
            emission_hbm.at[pl.ds(row0 + g * G, G), :], buf.at[b], sems[b]
        )

    def compute(g, b):
        acc = jnp.zeros((L,), jnp.int32)
        for r in range(G):
            init = ()
            for _ in range(NQ):
                init = init + (
                    jnp.full((L,), -jnp.inf, jnp.float32),
                    jnp.zeros((L,), jnp.int32),
                )

            def kbody(k, carry):
                ksplat = jnp.full((L,), k, jnp.int32)
                out = ()
                for q in range(NQ):
                    mv, mk = carry[2 * q], carry[2 * q + 1]
                    v = buf[b, r, pl.ds(k * L + q * QW, L)]
                    take = v > mv
                    out = out + (
                        jnp.where(take, v, mv),
                        jnp.where(take, ksplat, mk),
                    )
                return out

            carry = plsc.parallel_loop(0, QW // L, unroll=4, carry=init)(kbody)
            mv = carry[0]
            mi = (carry[1] << 4) + liota
            for q in range(1, NQ):
                v = carry[2 * q]
                i = (carry[2 * q + 1] << 4) + liota + q * QW
                take = v > mv
                mv = jnp.where(take, v, mv)
                mi = jnp.where(take, i, mi)
            m = jax.lax.reduce_max(mv, axes=(0,))
            masked = jnp.where(mv == m, mi, COLS)
            best = jax.lax.reduce_min(masked, axes=(0,))
            acc = jnp.where(liota == r, jnp.full((L,), best, jnp.int32), acc)
        outv[pl.ds(g * G, G)] = acc

    chunk_dma(0, 0).start()

    def outer(i, _):
        g0 = 2 * i
        chunk_dma(g0 + 1, 1).start()
        chunk_dma(g0, 0).wait()
        compute(g0, 0)

        @pl.when(g0 + 2 < NGS)
        def _():
            chunk_dma(g0 + 2, 0).start()

        chunk_dma(g0 + 1, 1).wait()
        compute(g0 + 1, 1)
        return 0

    lax.fori_loop(0, NGS // 2, outer, 0)
    pltpu.sync_copy(outv, out_hbm.at[pl.ds(row0, RPW)])


BLK = 256
NBLK = ROWS_TC // BLK
NBUF = 4


def _tc_body(em_hbm, o_hbm, buf, ov, osem, *sems):
    def blk_dma(g, b):
        return pltpu.make_async_copy(
            em_hbm.at[pl.ds(ROWS_SC + g * BLK, BLK), :], buf.at[b], sems[b]
        )

    for b in range(NBUF - 1):
        blk_dma(b, b).start()

    def compute(g, b):
        nsl = COLS // 128
        xs = [buf[b, :, pl.ds(j * 128, 128)] for j in range(nsl)]
        m = xs[0]
        for j in range(1, nsl):
            m = jnp.maximum(m, xs[j])
        m = jnp.max(m, axis=-1, keepdims=True)
        li = lax.broadcasted_iota(jnp.int32, (BLK, 128), 1)
        cand = jnp.where(xs[0] == m, li, COLS)
        for j in range(1, nsl):
            cand = jnp.minimum(cand, jnp.where(xs[j] == m, li + j * 128, COLS))
        ov[pl.ds(g * BLK, BLK)] = jnp.min(cand, axis=-1)

    def outer(i, _):
        g0 = NBUF * i
        for b in range(NBUF):
            g = g0 + b

            @pl.when(g + NBUF - 1 < NBLK)
            def _():
                blk_dma(g + NBUF - 1, (b + NBUF - 1) % NBUF).start()

            blk_dma(g, b).wait()
            compute(g, b)
        return 0

    lax.fori_loop(0, NBLK // NBUF, outer, 0)
    pltpu.make_async_copy(ov, o_hbm, osem).start()
    pltpu.make_async_copy(ov, o_hbm, osem).wait()


_argmax_tc = pl.pallas_call(
    _tc_body,
    in_specs=[pl.BlockSpec(memory_space=pl.ANY)],
    out_specs=pl.BlockSpec(memory_space=pl.ANY),
    out_shape=jax.ShapeDtypeStruct((ROWS_TC,), jnp.int32),
    scratch_shapes=[
        pltpu.VMEM((NBUF, BLK, COLS), jnp.float32),
        pltpu.VMEM((ROWS_TC,), jnp.int32),
        pltpu.SemaphoreType.DMA,
    ]
    + [pltpu.SemaphoreType.DMA] * NBUF,
)


def kernel(emission, to_string):
    del to_string
    tc_out = _argmax_tc(emission)
    sc_out = _argmax_sc(emission)
    return jnp.concatenate([sc_out, tc_out])
